# baseline (speedup 1.0000x reference)
import functools
import jax
import jax.numpy as jnp
from jax import lax
from jax.experimental import pallas as pl
from jax.experimental.pallas import tpu as pltpu
from jax.experimental.pallas import tpu_sc as plsc

_N, _U, _I, _E = 1024, 26, 26, 64
_NW = 32
_BPW = _N // _NW
_ROWS = _U * _I


def _sc_body(xu_hbm, xi_hbm, out_hbm, xu_v, xi_v, buf, sem, st_):
    nc = 2
    wid = lax.axis_index("s") * nc + lax.axis_index("c")
    b0 = wid * _BPW

    def batch_body(t, _):
        b = b0 + t
        t8 = lax.rem(t, 4)

        @pl.when(t8 == 0)
        def _stage4():
            cu = pltpu.async_copy(xu_hbm.at[pl.ds(b, 4)], xu_v, st_)
            ci = pltpu.async_copy(xi_hbm.at[pl.ds(b, 4)], xi_v, st_)
            cu.wait()
            ci.wait()
        for half in range(2):
            jbase = 13 * half
            items = [
                xi_v[t8, jbase + jj, pl.ds(16 * k, 16)]
                for jj in range(13)
                for k in range(4)
            ]

            def ibody(i, _, jbase=jbase, items=items, t8=t8):
                base = 26 * i + jbase
                u = [xu_v[t8, i, pl.ds(16 * k, 16)] for k in range(4)]
                for jj in range(13):
                    for k in range(4):
                        buf[base + jj, pl.ds(16 * k, 16)] = u[k]
                    for k in range(4):
                        buf[base + jj, pl.ds(64 + 16 * k, 16)] = items[4 * jj + k]
                return None

            lax.fori_loop(0, _U, ibody, None)

        pltpu.async_copy(buf, out_hbm.at[b], sem).wait()
        return None

    lax.fori_loop(0, _BPW, batch_body, None)


@jax.jit
def kernel(x_user, x_item):
    n, u, e = x_user.shape
    i = x_item.shape[1]
    mesh = plsc.VectorSubcoreMesh(core_axis_name="c", subcore_axis_name="s")
    f = functools.partial(
        pl.kernel,
        mesh=mesh,
        out_type=jax.ShapeDtypeStruct((n, u * i, 2 * e), jnp.float32),
        scratch_types=[
            pltpu.VMEM((4, u, e), jnp.float32),
            pltpu.VMEM((4, i, e), jnp.float32),
            pltpu.VMEM((u * i, 2 * e), jnp.float32),
            pltpu.SemaphoreType.DMA,
            pltpu.SemaphoreType.DMA,
        ],
    )(_sc_body)
    return f(x_user, x_item)


# speedup vs baseline: 1.0174x; 1.0174x over previous
import functools
import jax
import jax.numpy as jnp
from jax import lax
from jax.experimental import pallas as pl
from jax.experimental.pallas import tpu as pltpu
from jax.experimental.pallas import tpu_sc as plsc

_N, _U, _I, _E = 1024, 26, 26, 64
_NW = 32
_BPW = _N // _NW
_ROWS = _U * _I


def _sc_body(xu_hbm, xi_hbm, out_hbm, xu_v, xi_v, buf, sem, st_):
    nc = 2
    wid = lax.axis_index("s") * nc + lax.axis_index("c")
    b0 = wid * _BPW

    def batch_body(t, _):
        b = b0 + t
        t8 = lax.rem(t, 4)

        @pl.when(t == 0)
        def _stage_first():
            cu = pltpu.async_copy(xu_hbm.at[pl.ds(b, 4)], xu_v, st_)
            ci = pltpu.async_copy(xi_hbm.at[pl.ds(b, 4)], xi_v, st_)
            cu.wait()
            ci.wait()

        @pl.when((t8 == 0) & (t > 0))
        def _drain_prefetch():
            pltpu.make_async_copy(xu_hbm.at[pl.ds(b0, 4)], xu_v, st_).wait()
            pltpu.make_async_copy(xi_hbm.at[pl.ds(b0, 4)], xi_v, st_).wait()
        for half in range(2):
            jbase = 13 * half
            items = [
                xi_v[t8, jbase + jj, pl.ds(16 * k, 16)]
                for jj in range(13)
                for k in range(4)
            ]

            def ibody(i, _, jbase=jbase, items=items, t8=t8):
                base = 26 * i + jbase
                u = [xu_v[t8, i, pl.ds(16 * k, 16)] for k in range(4)]
                for jj in range(13):
                    for k in range(4):
                        buf[base + jj, pl.ds(16 * k, 16)] = u[k]
                    for k in range(4):
                        buf[base + jj, pl.ds(64 + 16 * k, 16)] = items[4 * jj + k]
                return None

            lax.fori_loop(0, _U, ibody, None)

        copy = pltpu.async_copy(buf, out_hbm.at[b], sem)

        @pl.when((t8 == 3) & (t < _BPW - 1))
        def _prefetch():
            pltpu.async_copy(xu_hbm.at[pl.ds(b + 1, 4)], xu_v, st_)
            pltpu.async_copy(xi_hbm.at[pl.ds(b + 1, 4)], xi_v, st_)

        copy.wait()
        return None

    lax.fori_loop(0, _BPW, batch_body, None)


@jax.jit
def kernel(x_user, x_item):
    n, u, e = x_user.shape
    i = x_item.shape[1]
    mesh = plsc.VectorSubcoreMesh(core_axis_name="c", subcore_axis_name="s")
    f = functools.partial(
        pl.kernel,
        mesh=mesh,
        out_type=jax.ShapeDtypeStruct((n, u * i, 2 * e), jnp.float32),
        scratch_types=[
            pltpu.VMEM((4, u, e), jnp.float32),
            pltpu.VMEM((4, i, e), jnp.float32),
            pltpu.VMEM((u * i, 2 * e), jnp.float32),
            pltpu.SemaphoreType.DMA,
            pltpu.SemaphoreType.DMA,
        ],
    )(_sc_body)
    return f(x_user, x_item)
